# baseline (speedup 1.0000x reference)
import jax
import jax.numpy as jnp
from jax import lax
from jax.experimental import pallas as pl
from jax.experimental.pallas import tpu as pltpu
from jax.experimental.pallas import tpu_sc as plsc

S = 2048
D = 768
FF = 3072
E = 8
T = 128
NPAD = 3072
NB = NPAD // T
NW = 32
CHUNK = S // NW


def _gate_body(x_ref, gw_ref, gb_ref, pos_ref, w_ref, bmap_ref, bact_ref,
               loss_ref, oh_ref):
    x = x_ref[...]
    logits = jnp.dot(x, gw_ref[...], preferred_element_type=jnp.float32)
    logits = logits + gb_ref[...]
    m = jnp.max(logits, axis=1, keepdims=True)
    ex = jnp.exp(logits - m)
    probs = ex / jnp.sum(ex, axis=1, keepdims=True)
    w = jnp.max(probs, axis=1, keepdims=True)
    eidx = lax.broadcasted_iota(jnp.int32, (S, E), 1)
    idx = jnp.min(jnp.where(probs == w, eidx, E), axis=1, keepdims=True)
    oh = (eidx == idx).astype(jnp.float32)
    oh_ref[...] = oh
    w_ref[...] = jnp.broadcast_to(w, (S, 128))

    counts = jnp.sum(oh, axis=0, keepdims=True)
    ssum = jnp.sum(oh * w, axis=0, keepdims=True)
    usage = ssum / (counts + 1e-8)
    loss_ref[...] = jnp.sum((usage - 1.0 / E) ** 2).reshape(1, 1)

    nblk = jnp.floor((counts + (T - 1)) / T)
    ublk = (lax.broadcasted_iota(jnp.int32, (E, E), 0) <
            lax.broadcasted_iota(jnp.int32, (E, E), 1)).astype(jnp.float32)
    blk_start = jnp.dot(nblk, ublk, preferred_element_type=jnp.float32)
    row_off = blk_start * T

    bi = lax.broadcasted_iota(jnp.int32, (NB, E), 0).astype(jnp.float32)
    ge = (bi >= blk_start).astype(jnp.float32)
    bmap_ref[...] = (jnp.sum(ge, axis=1, keepdims=True) - 1.0).astype(jnp.int32)
    nblk_tot = jnp.sum(nblk)
    bact_ref[...] = (bi[:, 0:1] < nblk_tot).astype(jnp.int32)

    tril = (lax.broadcasted_iota(jnp.int32, (T, T), 1) <=
            lax.broadcasted_iota(jnp.int32, (T, T), 0)).astype(jnp.float32)

    def body(c, base):
        ohc = oh_ref[pl.ds(c * T, T), :]
        inc = jnp.dot(tril, ohc, preferred_element_type=jnp.float32)
        posc = jnp.sum(ohc * (row_off + base + inc - 1.0), axis=1,
                       keepdims=True)
        pos_ref[pl.ds(c * T, T), :] = posc.astype(jnp.int32)
        return base + jnp.sum(ohc, axis=0, keepdims=True)

    lax.fori_loop(0, S // T, body, jnp.zeros((1, E), jnp.float32))


def _gate_call(x2, gate_W, gate_b):
    return pl.pallas_call(
        _gate_body,
        out_shape=(
            jax.ShapeDtypeStruct((S, 1), jnp.int32),
            jax.ShapeDtypeStruct((S, 128), jnp.float32),
            jax.ShapeDtypeStruct((NB, 1), jnp.int32),
            jax.ShapeDtypeStruct((NB, 1), jnp.int32),
            jax.ShapeDtypeStruct((1, 1), jnp.float32),
        ),
        scratch_shapes=[pltpu.VMEM((S, E), jnp.float32)],
    )(x2, gate_W, gate_b.reshape(1, E))


def _dispatch_body(pos_hbm, w_hbm, x_hbm, xs_hbm, ws_hbm,
                   posv, xv, wv, sem, sem2, sem3):
    wid = lax.axis_index("s") * 2 + lax.axis_index("c")
    base = wid * CHUNK
    cp = pltpu.async_copy(pos_hbm.at[pl.ds(base, CHUNK)], posv, sem)
    cxl = pltpu.async_copy(x_hbm.at[pl.ds(base, CHUNK)], xv, sem2)
    cwl = pltpu.async_copy(w_hbm.at[pl.ds(base, CHUNK)], wv, sem3)
    cp.wait()
    cxl.wait()
    cwl.wait()
    cx = pltpu.async_copy(xv, xs_hbm.at[posv], sem)
    cw = pltpu.async_copy(wv, ws_hbm.at[posv], sem2)
    cx.wait()
    cw.wait()


def _dispatch_call(pos, w16, x2):
    mesh = plsc.VectorSubcoreMesh(core_axis_name="c", subcore_axis_name="s")
    return pl.kernel(
        _dispatch_body,
        out_type=(
            jax.ShapeDtypeStruct((NPAD, D), jnp.float32),
            jax.ShapeDtypeStruct((NPAD, 128), jnp.float32),
        ),
        mesh=mesh,
        scratch_types=[
            pltpu.VMEM((CHUNK,), jnp.int32),
            pltpu.VMEM((CHUNK, D), jnp.float32),
            pltpu.VMEM((CHUNK, 128), jnp.float32),
            pltpu.SemaphoreType.DMA,
            pltpu.SemaphoreType.DMA,
            pltpu.SemaphoreType.DMA,
        ],
    )(pos, w16, x2)


def _ffn_body(bm, ba, xs_ref, w1_ref, w2_ref, b1_ref, b2_ref, ow_ref, ob_ref,
              ws_ref, ys_ref):
    b = pl.program_id(0)

    @pl.when(ba[b] == 1)
    def _():
        xs = xs_ref[...].astype(jnp.bfloat16)
        w1 = w1_ref[0].astype(jnp.bfloat16)
        w2 = w2_ref[0].astype(jnp.bfloat16)
        h1 = jnp.dot(xs, w1, preferred_element_type=jnp.float32) + b1_ref[0]
        h2 = jnp.dot(xs, w2, preferred_element_type=jnp.float32) + b2_ref[0]
        g = 0.5 * h2 * (1.0 + lax.erf(h2 * (2.0 ** -0.5)))
        act = (h1 * g).astype(jnp.bfloat16)
        ow = ow_ref[0].astype(jnp.bfloat16)
        y = jnp.dot(act, ow, preferred_element_type=jnp.float32)
        ys_ref[...] = (y + ob_ref[0]) * ws_ref[0][:, 0:1]


def _ffn_call(bmap, bact, xs, fc_W, fc_b3, out_W, out_b3, ws3):
    grid_spec = pltpu.PrefetchScalarGridSpec(
        num_scalar_prefetch=2,
        grid=(NB,),
        in_specs=[
            pl.BlockSpec((T, D), lambda b, bm, ba: (b, 0)),
            pl.BlockSpec((1, D, FF), lambda b, bm, ba: (bm[b], 0, 0)),
            pl.BlockSpec((1, D, FF), lambda b, bm, ba: (bm[b], 0, 1)),
            pl.BlockSpec((1, 1, FF), lambda b, bm, ba: (bm[b], 0, 0)),
            pl.BlockSpec((1, 1, FF), lambda b, bm, ba: (bm[b], 0, 1)),
            pl.BlockSpec((1, FF, D), lambda b, bm, ba: (bm[b], 0, 0)),
            pl.BlockSpec((1, 1, D), lambda b, bm, ba: (bm[b], 0, 0)),
            pl.BlockSpec((1, T, 128), lambda b, bm, ba: (b, 0, 0)),
        ],
        out_specs=pl.BlockSpec((T, D), lambda b, bm, ba: (b, 0)),
    )
    return pl.pallas_call(
        _ffn_body,
        grid_spec=grid_spec,
        out_shape=jax.ShapeDtypeStruct((NPAD, D), jnp.float32),
        compiler_params=pltpu.CompilerParams(
            dimension_semantics=("arbitrary",)),
    )(bmap, bact, xs, fc_W, fc_W, fc_b3, fc_b3, out_W, out_b3, ws3)


def _combine_body(pos_hbm, ys_hbm, out_hbm, posv, rows, sem):
    wid = lax.axis_index("s") * 2 + lax.axis_index("c")
    base = wid * CHUNK
    pltpu.sync_copy(pos_hbm.at[pl.ds(base, CHUNK)], posv)
    pltpu.async_copy(ys_hbm.at[posv], rows, sem).wait()
    pltpu.sync_copy(rows, out_hbm.at[pl.ds(base, CHUNK)])


def _combine_call(pos, ys):
    mesh = plsc.VectorSubcoreMesh(core_axis_name="c", subcore_axis_name="s")
    return pl.kernel(
        _combine_body,
        out_type=jax.ShapeDtypeStruct((S, D), jnp.float32),
        mesh=mesh,
        scratch_types=[
            pltpu.VMEM((CHUNK,), jnp.int32),
            pltpu.VMEM((CHUNK, D), jnp.float32),
            pltpu.SemaphoreType.DMA,
        ],
    )(pos, ys)


def kernel(x, gate_W, gate_b, fc_W, fc_b, out_W, out_b):
    x2 = x.reshape(S, D)
    pos2, w16, bmap2, bact2, uloss = _gate_call(x2, gate_W, gate_b)
    pos = pos2.reshape(S)
    bmap = bmap2.reshape(NB)
    bact = bact2.reshape(NB)
    xs, ws = _dispatch_call(pos, w16, x2)
    ys = _ffn_call(bmap, bact, xs, fc_W, fc_b.reshape(E, 1, 2 * FF),
                   out_W, out_b.reshape(E, 1, D), ws.reshape(NB, T, 128))
    out = _combine_call(pos, ys)
    return out.reshape(1, S, D), uloss.reshape(())


# speedup vs baseline: 1.0076x; 1.0076x over previous
import jax
import jax.numpy as jnp
from jax import lax
from jax.experimental import pallas as pl
from jax.experimental.pallas import tpu as pltpu
from jax.experimental.pallas import tpu_sc as plsc

S = 2048
D = 768
FF = 3072
E = 8
T = 128
NPAD = 3072
NB = NPAD // T
NW = 32
CHUNK = S // NW


def _gate_body(x_ref, gw_ref, gb_ref, pos_ref, w_ref, bmap_ref, bact_ref,
               loss_ref, oh_ref):
    x = x_ref[...]
    logits = jnp.dot(x, gw_ref[...], preferred_element_type=jnp.float32)
    logits = logits + gb_ref[...]
    m = jnp.max(logits, axis=1, keepdims=True)
    ex = jnp.exp(logits - m)
    probs = ex / jnp.sum(ex, axis=1, keepdims=True)
    w = jnp.max(probs, axis=1, keepdims=True)
    eidx = lax.broadcasted_iota(jnp.int32, (S, E), 1)
    idx = jnp.min(jnp.where(probs == w, eidx, E), axis=1, keepdims=True)
    oh = (eidx == idx).astype(jnp.float32)
    oh_ref[...] = oh
    w_ref[...] = jnp.broadcast_to(w, (S, 128))

    counts = jnp.sum(oh, axis=0, keepdims=True)
    ssum = jnp.sum(oh * w, axis=0, keepdims=True)
    usage = ssum / (counts + 1e-8)
    loss_ref[...] = jnp.sum((usage - 1.0 / E) ** 2).reshape(1, 1)

    nblk = jnp.floor((counts + (T - 1)) / T)
    ublk = (lax.broadcasted_iota(jnp.int32, (E, E), 0) <
            lax.broadcasted_iota(jnp.int32, (E, E), 1)).astype(jnp.float32)
    blk_start = jnp.dot(nblk, ublk, preferred_element_type=jnp.float32)
    row_off = blk_start * T

    bi = lax.broadcasted_iota(jnp.int32, (NB, E), 0).astype(jnp.float32)
    ge = (bi >= blk_start).astype(jnp.float32)
    bmap_ref[...] = (jnp.sum(ge, axis=1, keepdims=True) - 1.0).astype(jnp.int32)
    nblk_tot = jnp.sum(nblk)
    bact_ref[...] = (bi[:, 0:1] < nblk_tot).astype(jnp.int32)

    tril = (lax.broadcasted_iota(jnp.int32, (T, T), 1) <=
            lax.broadcasted_iota(jnp.int32, (T, T), 0)).astype(jnp.float32)

    def body(c, base):
        ohc = oh_ref[pl.ds(c * T, T), :]
        inc = jnp.dot(tril, ohc, preferred_element_type=jnp.float32)
        posc = jnp.sum(ohc * (row_off + base + inc - 1.0), axis=1,
                       keepdims=True)
        pos_ref[pl.ds(c * T, T), :] = posc.astype(jnp.int32)
        return base + jnp.sum(ohc, axis=0, keepdims=True)

    lax.fori_loop(0, S // T, body, jnp.zeros((1, E), jnp.float32))


def _gate_call(x2, gate_W, gate_b):
    return pl.pallas_call(
        _gate_body,
        out_shape=(
            jax.ShapeDtypeStruct((S, 1), jnp.int32),
            jax.ShapeDtypeStruct((S, 128), jnp.float32),
            jax.ShapeDtypeStruct((NB, 1), jnp.int32),
            jax.ShapeDtypeStruct((NB, 1), jnp.int32),
            jax.ShapeDtypeStruct((1, 1), jnp.float32),
        ),
        scratch_shapes=[pltpu.VMEM((S, E), jnp.float32)],
    )(x2, gate_W, gate_b.reshape(1, E))


def _dispatch_body(pos_hbm, w_hbm, x_hbm, xs_hbm, ws_hbm,
                   posv, xv, wv, sem, sem2, sem3):
    wid = lax.axis_index("s") * 2 + lax.axis_index("c")
    base = wid * CHUNK
    cp = pltpu.async_copy(pos_hbm.at[pl.ds(base, CHUNK)], posv, sem)
    cxl = pltpu.async_copy(x_hbm.at[pl.ds(base, CHUNK)], xv, sem2)
    cwl = pltpu.async_copy(w_hbm.at[pl.ds(base, CHUNK)], wv, sem3)
    cp.wait()
    cxl.wait()
    cwl.wait()
    cx = pltpu.async_copy(xv, xs_hbm.at[posv], sem)
    cw = pltpu.async_copy(wv, ws_hbm.at[posv], sem2)
    cx.wait()
    cw.wait()


def _dispatch_call(pos, w16, x2):
    mesh = plsc.VectorSubcoreMesh(core_axis_name="c", subcore_axis_name="s")
    return pl.kernel(
        _dispatch_body,
        out_type=(
            jax.ShapeDtypeStruct((NPAD, D), jnp.float32),
            jax.ShapeDtypeStruct((NPAD, 128), jnp.float32),
        ),
        mesh=mesh,
        scratch_types=[
            pltpu.VMEM((CHUNK,), jnp.int32),
            pltpu.VMEM((CHUNK, D), jnp.float32),
            pltpu.VMEM((CHUNK, 128), jnp.float32),
            pltpu.SemaphoreType.DMA,
            pltpu.SemaphoreType.DMA,
            pltpu.SemaphoreType.DMA,
        ],
    )(pos, w16, x2)


def _ffn_body(bm, ba, xs_ref, w1_ref, w2_ref, b1_ref, b2_ref, ow_ref, ob_ref,
              ws_ref, ys_ref):
    b = pl.program_id(0)

    @pl.when(ba[b] == 1)
    def _():
        xs = xs_ref[...].astype(jnp.bfloat16)
        w1 = w1_ref[0].astype(jnp.bfloat16)
        w2 = w2_ref[0].astype(jnp.bfloat16)
        h1 = jnp.dot(xs, w1, preferred_element_type=jnp.float32) + b1_ref[0]
        h2 = jnp.dot(xs, w2, preferred_element_type=jnp.float32) + b2_ref[0]
        g = 0.5 * h2 * (1.0 + lax.erf(h2 * (2.0 ** -0.5)))
        act = (h1 * g).astype(jnp.bfloat16)
        ow = ow_ref[0].astype(jnp.bfloat16)
        y = jnp.dot(act, ow, preferred_element_type=jnp.float32)
        ys_ref[...] = (y + ob_ref[0]) * ws_ref[0][:, 0:1]


def _ffn_call(bmap, bact, xs, fc_W, fc_b3, out_W, out_b3, ws3):
    grid_spec = pltpu.PrefetchScalarGridSpec(
        num_scalar_prefetch=2,
        grid=(NB,),
        in_specs=[
            pl.BlockSpec((T, D), lambda b, bm, ba: (ba[b] * b, 0)),
            pl.BlockSpec((1, D, FF), lambda b, bm, ba: (bm[b], 0, 0)),
            pl.BlockSpec((1, D, FF), lambda b, bm, ba: (bm[b], 0, 1)),
            pl.BlockSpec((1, 1, FF), lambda b, bm, ba: (bm[b], 0, 0)),
            pl.BlockSpec((1, 1, FF), lambda b, bm, ba: (bm[b], 0, 1)),
            pl.BlockSpec((1, FF, D), lambda b, bm, ba: (bm[b], 0, 0)),
            pl.BlockSpec((1, 1, D), lambda b, bm, ba: (bm[b], 0, 0)),
            pl.BlockSpec((1, T, 128), lambda b, bm, ba: (ba[b] * b, 0, 0)),
        ],
        out_specs=pl.BlockSpec((T, D), lambda b, bm, ba: (b, 0)),
    )
    return pl.pallas_call(
        _ffn_body,
        grid_spec=grid_spec,
        out_shape=jax.ShapeDtypeStruct((NPAD, D), jnp.float32),
        compiler_params=pltpu.CompilerParams(
            dimension_semantics=("arbitrary",)),
    )(bmap, bact, xs, fc_W, fc_W, fc_b3, fc_b3, out_W, out_b3, ws3)


def _combine_body(pos_hbm, ys_hbm, out_hbm, posv, rows, sem):
    wid = lax.axis_index("s") * 2 + lax.axis_index("c")
    base = wid * CHUNK
    pltpu.sync_copy(pos_hbm.at[pl.ds(base, CHUNK)], posv)
    pltpu.async_copy(ys_hbm.at[posv], rows, sem).wait()
    pltpu.sync_copy(rows, out_hbm.at[pl.ds(base, CHUNK)])


def _combine_call(pos, ys):
    mesh = plsc.VectorSubcoreMesh(core_axis_name="c", subcore_axis_name="s")
    return pl.kernel(
        _combine_body,
        out_type=jax.ShapeDtypeStruct((S, D), jnp.float32),
        mesh=mesh,
        scratch_types=[
            pltpu.VMEM((CHUNK,), jnp.int32),
            pltpu.VMEM((CHUNK, D), jnp.float32),
            pltpu.SemaphoreType.DMA,
        ],
    )(pos, ys)


def kernel(x, gate_W, gate_b, fc_W, fc_b, out_W, out_b):
    x2 = x.reshape(S, D)
    pos2, w16, bmap2, bact2, uloss = _gate_call(x2, gate_W, gate_b)
    pos = pos2.reshape(S)
    bmap = bmap2.reshape(NB)
    bact = bact2.reshape(NB)
    xs, ws = _dispatch_call(pos, w16, x2)
    ys = _ffn_call(bmap, bact, xs, fc_W, fc_b.reshape(E, 1, 2 * FF),
                   out_W, out_b.reshape(E, 1, D), ws.reshape(NB, T, 128))
    out = _combine_call(pos, ys)
    return out.reshape(1, S, D), uloss.reshape(())


# speedup vs baseline: 1.0097x; 1.0020x over previous
import jax
import jax.numpy as jnp
from jax import lax
from jax.experimental import pallas as pl
from jax.experimental.pallas import tpu as pltpu
from jax.experimental.pallas import tpu_sc as plsc

S = 2048
D = 768
FF = 3072
E = 8
T = 128
NPAD = 3072
NB = NPAD // T
NW = 32
CHUNK = S // NW


def _gate_body(x_ref, gw_ref, gb_ref, pos_ref, w_ref, bmap_ref, bact_ref,
               loss_ref, oh_ref):
    x = x_ref[...]
    logits = jnp.dot(x, gw_ref[...], preferred_element_type=jnp.float32)
    logits = logits + gb_ref[...]
    m = jnp.max(logits, axis=1, keepdims=True)
    ex = jnp.exp(logits - m)
    probs = ex / jnp.sum(ex, axis=1, keepdims=True)
    w = jnp.max(probs, axis=1, keepdims=True)
    eidx = lax.broadcasted_iota(jnp.int32, (S, E), 1)
    idx = jnp.min(jnp.where(probs == w, eidx, E), axis=1, keepdims=True)
    oh = (eidx == idx).astype(jnp.float32)
    oh_ref[...] = oh
    w_ref[...] = jnp.broadcast_to(w, (S, 128))

    counts = jnp.sum(oh, axis=0, keepdims=True)
    ssum = jnp.sum(oh * w, axis=0, keepdims=True)
    usage = ssum / (counts + 1e-8)
    loss_ref[...] = jnp.sum((usage - 1.0 / E) ** 2).reshape(1, 1)

    nblk = jnp.floor((counts + (T - 1)) / T)
    ublk = (lax.broadcasted_iota(jnp.int32, (E, E), 0) <
            lax.broadcasted_iota(jnp.int32, (E, E), 1)).astype(jnp.float32)
    blk_start = jnp.dot(nblk, ublk, preferred_element_type=jnp.float32)
    row_off = blk_start * T

    bi = lax.broadcasted_iota(jnp.int32, (NB, E), 0).astype(jnp.float32)
    ge = (bi >= blk_start).astype(jnp.float32)
    bmap_ref[...] = (jnp.sum(ge, axis=1, keepdims=True) - 1.0).astype(jnp.int32)
    nblk_tot = jnp.sum(nblk)
    bact_ref[...] = (bi[:, 0:1] < nblk_tot).astype(jnp.int32)

    tril = (lax.broadcasted_iota(jnp.int32, (T, T), 1) <=
            lax.broadcasted_iota(jnp.int32, (T, T), 0)).astype(jnp.float32)

    def body(c, base):
        ohc = oh_ref[pl.ds(c * T, T), :]
        inc = jnp.dot(tril, ohc, preferred_element_type=jnp.float32)
        posc = jnp.sum(ohc * (row_off + base + inc - 1.0), axis=1,
                       keepdims=True)
        pos_ref[pl.ds(c * T, T), :] = posc.astype(jnp.int32)
        return base + jnp.sum(ohc, axis=0, keepdims=True)

    lax.fori_loop(0, S // T, body, jnp.zeros((1, E), jnp.float32))


def _gate_call(x2, gate_W, gate_b):
    return pl.pallas_call(
        _gate_body,
        out_shape=(
            jax.ShapeDtypeStruct((S, 1), jnp.int32),
            jax.ShapeDtypeStruct((S, 128), jnp.float32),
            jax.ShapeDtypeStruct((NB, 1), jnp.int32),
            jax.ShapeDtypeStruct((NB, 1), jnp.int32),
            jax.ShapeDtypeStruct((1, 1), jnp.float32),
        ),
        scratch_shapes=[pltpu.VMEM((S, E), jnp.float32)],
    )(x2, gate_W, gate_b.reshape(1, E))


def _dispatch_body(pos_hbm, w_hbm, x_hbm, xs_hbm, ws_hbm,
                   posv, xv, wv, sem, sem2, sem3):
    wid = lax.axis_index("s") * 2 + lax.axis_index("c")
    base = wid * CHUNK
    cp = pltpu.async_copy(pos_hbm.at[pl.ds(base, CHUNK)], posv, sem)
    cxl = pltpu.async_copy(x_hbm.at[pl.ds(base, CHUNK)], xv, sem2)
    cwl = pltpu.async_copy(w_hbm.at[pl.ds(base, CHUNK)], wv, sem3)
    cp.wait()
    cxl.wait()
    cwl.wait()
    cx = pltpu.async_copy(xv, xs_hbm.at[posv], sem)
    cw = pltpu.async_copy(wv, ws_hbm.at[posv], sem2)
    cx.wait()
    cw.wait()


def _dispatch_call(pos, w16, x2):
    mesh = plsc.VectorSubcoreMesh(core_axis_name="c", subcore_axis_name="s")
    return pl.kernel(
        _dispatch_body,
        out_type=(
            jax.ShapeDtypeStruct((NPAD, D), jnp.float32),
            jax.ShapeDtypeStruct((NPAD, 128), jnp.float32),
        ),
        mesh=mesh,
        scratch_types=[
            pltpu.VMEM((CHUNK,), jnp.int32),
            pltpu.VMEM((CHUNK, D), jnp.float32),
            pltpu.VMEM((CHUNK, 128), jnp.float32),
            pltpu.SemaphoreType.DMA,
            pltpu.SemaphoreType.DMA,
            pltpu.SemaphoreType.DMA,
        ],
    )(pos, w16, x2)


def _ffn_body(bm, ba, xs_ref, w1_ref, w2_ref, b1_ref, b2_ref, ow_ref, ob_ref,
              ws_ref, ys_ref):
    b = pl.program_id(0)

    @pl.when(ba[b] == 1)
    def _():
        xs = xs_ref[...].astype(jnp.bfloat16)
        w1 = w1_ref[0].astype(jnp.bfloat16)
        w2 = w2_ref[0].astype(jnp.bfloat16)
        h1 = jnp.dot(xs, w1, preferred_element_type=jnp.float32) + b1_ref[0]
        h2 = jnp.dot(xs, w2, preferred_element_type=jnp.float32) + b2_ref[0]
        g = 0.5 * h2 * (1.0 + lax.erf(h2 * (2.0 ** -0.5)))
        act = (h1 * g).astype(jnp.bfloat16)
        ow = ow_ref[0].astype(jnp.bfloat16)
        y = jnp.dot(act, ow, preferred_element_type=jnp.float32)
        ys_ref[...] = (y + ob_ref[0]) * ws_ref[:, 0:1]


def _ffn_call(bmap, bact, xs, fc_W, fc_b3, out_W, out_b3, ws3):
    grid_spec = pltpu.PrefetchScalarGridSpec(
        num_scalar_prefetch=2,
        grid=(NB,),
        in_specs=[
            pl.BlockSpec((T, D), lambda b, bm, ba: (ba[b] * b, 0)),
            pl.BlockSpec((1, D, FF), lambda b, bm, ba: (bm[b], 0, 0)),
            pl.BlockSpec((1, D, FF), lambda b, bm, ba: (bm[b], 0, 1)),
            pl.BlockSpec((1, 1, FF), lambda b, bm, ba: (bm[b], 0, 0)),
            pl.BlockSpec((1, 1, FF), lambda b, bm, ba: (bm[b], 0, 1)),
            pl.BlockSpec((1, FF, D), lambda b, bm, ba: (bm[b], 0, 0)),
            pl.BlockSpec((1, 1, D), lambda b, bm, ba: (bm[b], 0, 0)),
            pl.BlockSpec((T, 128), lambda b, bm, ba: (ba[b] * b, 0)),
        ],
        out_specs=pl.BlockSpec((T, D), lambda b, bm, ba: (b, 0)),
    )
    return pl.pallas_call(
        _ffn_body,
        grid_spec=grid_spec,
        out_shape=jax.ShapeDtypeStruct((NPAD, D), jnp.float32),
        compiler_params=pltpu.CompilerParams(
            dimension_semantics=("arbitrary",)),
    )(bmap, bact, xs, fc_W, fc_W, fc_b3, fc_b3, out_W, out_b3, ws3)


def _combine_body(pos_hbm, ys_hbm, out_hbm, posv, rows, sem):
    wid = lax.axis_index("s") * 2 + lax.axis_index("c")
    base = wid * CHUNK
    pltpu.sync_copy(pos_hbm.at[pl.ds(base, CHUNK)], posv)
    pltpu.async_copy(ys_hbm.at[posv], rows, sem).wait()
    pltpu.sync_copy(rows, out_hbm.at[pl.ds(base, CHUNK)])


def _combine_call(pos, ys):
    mesh = plsc.VectorSubcoreMesh(core_axis_name="c", subcore_axis_name="s")
    return pl.kernel(
        _combine_body,
        out_type=jax.ShapeDtypeStruct((S, D), jnp.float32),
        mesh=mesh,
        scratch_types=[
            pltpu.VMEM((CHUNK,), jnp.int32),
            pltpu.VMEM((CHUNK, D), jnp.float32),
            pltpu.SemaphoreType.DMA,
        ],
    )(pos, ys)


def kernel(x, gate_W, gate_b, fc_W, fc_b, out_W, out_b):
    x2 = x.reshape(S, D)
    pos2, w16, bmap2, bact2, uloss = _gate_call(x2, gate_W, gate_b)
    pos = pos2.reshape(S)
    bmap = bmap2.reshape(NB)
    bact = bact2.reshape(NB)
    xs, ws = _dispatch_call(pos, w16, x2)
    ys = _ffn_call(bmap, bact, xs, fc_W, fc_b.reshape(E, 1, 2 * FF),
                   out_W, out_b.reshape(E, 1, D), ws)
    out = _combine_call(pos, ys)
    return out.reshape(1, S, D), uloss.reshape(())
